# Initial kernel scaffold; baseline (speedup 1.0000x reference)
import jax
import jax.numpy as jnp
from jax.experimental import pallas as pl


def kernel(images, cls0, cls1, cls2, cls3, cls4, reg0, reg1, reg2, reg3, reg4, ctr0, ctr1, ctr2, ctr3, ctr4):
    raise NotImplementedError("write your pallas kernel here")


# speedup vs baseline: 1.1679x; 1.1679x over previous
import jax
import jax.numpy as jnp
from jax.experimental import pallas as pl

_NUM_CLASSES = 80
_FPN_STRIDES = (8.0, 16.0, 32.0, 64.0, 128.0)
_IMG = 512.0
_K = 1000
_NCAND = 5 * _K
_NPAD = 5120
_POST = 50
_NOUT = 64
_PRE_NMS_THRESH = 0.3
_NMS_THRESH = 0.45


def _score_kernel(cls_ref, ctr_ref, out_ref):
    cls_p = jax.nn.sigmoid(cls_ref[...])
    ctr_p = jax.nn.sigmoid(ctr_ref[...])
    out_ref[...] = jnp.where(cls_p > _PRE_NMS_THRESH, cls_p * ctr_p, 0.0)


def _scores(cls_f, ctr_f):
    return pl.pallas_call(
        _score_kernel,
        out_shape=jax.ShapeDtypeStruct(cls_f.shape, jnp.float32),
    )(cls_f, ctr_f)


def _nms_kernel(r_ref, loc_ref, v_ref, c_ref, bo_ref, so_ref, co_ref):
    r = r_ref[0]
    loc = loc_ref[0]
    v = v_ref[0]
    cf = c_ref[0]

    lane = jax.lax.broadcasted_iota(jnp.int32, (1, _NPAD), 1)

    stride = jnp.where(
        lane < 1 * _K, _FPN_STRIDES[0],
        jnp.where(lane < 2 * _K, _FPN_STRIDES[1],
                  jnp.where(lane < 3 * _K, _FPN_STRIDES[2],
                            jnp.where(lane < 4 * _K, _FPN_STRIDES[3],
                                      _FPN_STRIDES[4]))))
    hsz = _IMG / stride

    xf = jnp.floor(loc / hsz)
    yf = loc - xf * hsz
    ysel = 0.5 * stride + yf * stride
    xsel = 0.5 * stride + xf * stride

    d = jnp.exp(r) * stride
    hi = _IMG - 1.0
    x1 = jnp.clip(xsel - d[0:1], 0.0, hi)
    y1 = jnp.clip(ysel - d[1:2], 0.0, hi)
    x2 = jnp.clip(xsel + d[2:3], 0.0, hi)
    y2 = jnp.clip(ysel + d[3:4], 0.0, hi)

    valid = lane < _NCAND
    scores = jnp.where(valid, jnp.sqrt(jnp.maximum(v, 1e-12)), -1.0)

    off = cf * (2.0 * _IMG)
    x1o = x1 + off
    y1o = y1 + off
    x2o = x2 + off
    y2o = y2 + off
    area = (x2o - x1o) * (y2o - y1o)

    olane = jax.lax.broadcasted_iota(jnp.int32, (1, _NOUT), 1)
    zo = jnp.zeros((1, _NOUT), jnp.float32)

    def body(i, carry):
        s, ox1, oy1, ox2, oy2, osc, ocl = carry
        m = jnp.max(s)
        best = jnp.min(jnp.where(s == m, lane, _NPAD))
        msk = lane == best
        bcl = jnp.sum(jnp.where(msk, cf, 0.0))
        bsc = jnp.sum(jnp.where(msk, scores, 0.0))
        bx1 = jnp.sum(jnp.where(msk, x1o, 0.0))
        by1 = jnp.sum(jnp.where(msk, y1o, 0.0))
        bx2 = jnp.sum(jnp.where(msk, x2o, 0.0))
        by2 = jnp.sum(jnp.where(msk, y2o, 0.0))
        ix1 = jnp.maximum(bx1, x1o)
        iy1 = jnp.maximum(by1, y1o)
        ix2 = jnp.minimum(bx2, x2o)
        iy2 = jnp.minimum(by2, y2o)
        inter = jnp.maximum(ix2 - ix1, 0.0) * jnp.maximum(iy2 - iy1, 0.0)
        barea = (bx2 - bx1) * (by2 - by1)
        iou = inter / jnp.maximum(barea + area - inter, 1e-6)
        s = jnp.where(iou > _NMS_THRESH, -1.0, s)
        s = jnp.where(msk, -1.0, s)
        sel = olane == i
        boff = bcl * (2.0 * _IMG)
        ox1 = jnp.where(sel, bx1 - boff, ox1)
        oy1 = jnp.where(sel, by1 - boff, oy1)
        ox2 = jnp.where(sel, bx2 - boff, ox2)
        oy2 = jnp.where(sel, by2 - boff, oy2)
        osc = jnp.where(sel, bsc, osc)
        ocl = jnp.where(sel, bcl, ocl)
        return (s, ox1, oy1, ox2, oy2, osc, ocl)

    init = (scores, zo, zo, zo, zo, zo, zo)
    _, ox1, oy1, ox2, oy2, osc, ocl = jax.lax.fori_loop(0, _POST, body, init)

    bo_ref[0] = jnp.concatenate([ox1, oy1, ox2, oy2], axis=0)
    so_ref[0] = osc
    co_ref[0] = ocl


def kernel(images, cls0, cls1, cls2, cls3, cls4,
           reg0, reg1, reg2, reg3, reg4,
           ctr0, ctr1, ctr2, ctr3, ctr4):
    b = images.shape[0]
    cls_list = [cls0, cls1, cls2, cls3, cls4]
    reg_list = [reg0, reg1, reg2, reg3, reg4]
    ctr_list = [ctr0, ctr1, ctr2, ctr3, ctr4]

    rs, ls, vs, cs = [], [], [], []
    for lvl in range(5):
        cls = cls_list[lvl]
        _, C, h, w = cls.shape
        n = h * w
        cls_f = jnp.transpose(cls, (0, 3, 2, 1)).reshape(b, n, C)
        ctr_f = jnp.transpose(ctr_list[lvl], (0, 3, 2, 1)).reshape(b, n, 1)
        reg_f = jnp.transpose(reg_list[lvl], (0, 3, 2, 1)).reshape(b, n, 4)
        sc = _scores(cls_f, ctr_f).reshape(b, n * C)
        topv, topi = jax.lax.top_k(sc, _K)
        loc_idx = topi // C
        cls_idx = topi % C
        rsel = jnp.take_along_axis(reg_f, loc_idx[..., None], axis=1)
        rs.append(rsel)
        ls.append(loc_idx)
        vs.append(topv)
        cs.append(cls_idx)

    rsel = jnp.concatenate(rs, axis=1)
    loc = jnp.concatenate(ls, axis=1).astype(jnp.float32)
    topv = jnp.concatenate(vs, axis=1)
    clsf = jnp.concatenate(cs, axis=1).astype(jnp.float32)

    pad = _NPAD - _NCAND
    r4 = jnp.pad(jnp.transpose(rsel, (0, 2, 1)), ((0, 0), (0, 0), (0, pad)))
    locp = jnp.pad(loc, ((0, 0), (0, pad)))[:, None, :]
    vp = jnp.pad(topv, ((0, 0), (0, pad)))[:, None, :]
    cp = jnp.pad(clsf, ((0, 0), (0, pad)))[:, None, :]

    bo, so, co = pl.pallas_call(
        _nms_kernel,
        grid=(b,),
        in_specs=[
            pl.BlockSpec((1, 4, _NPAD), lambda i: (i, 0, 0)),
            pl.BlockSpec((1, 1, _NPAD), lambda i: (i, 0, 0)),
            pl.BlockSpec((1, 1, _NPAD), lambda i: (i, 0, 0)),
            pl.BlockSpec((1, 1, _NPAD), lambda i: (i, 0, 0)),
        ],
        out_specs=[
            pl.BlockSpec((1, 4, _NOUT), lambda i: (i, 0, 0)),
            pl.BlockSpec((1, 1, _NOUT), lambda i: (i, 0, 0)),
            pl.BlockSpec((1, 1, _NOUT), lambda i: (i, 0, 0)),
        ],
        out_shape=[
            jax.ShapeDtypeStruct((b, 4, _NOUT), jnp.float32),
            jax.ShapeDtypeStruct((b, 1, _NOUT), jnp.float32),
            jax.ShapeDtypeStruct((b, 1, _NOUT), jnp.float32),
        ],
    )(r4, locp, vp, cp)

    pb = jnp.transpose(bo, (0, 2, 1))[:, :_POST, :]
    ps = so[:, 0, :_POST]
    pc = co[:, 0, :_POST].astype(jnp.int32)
    return pb, ps, pc


# speedup vs baseline: 1.1850x; 1.0146x over previous
import jax
import jax.numpy as jnp
from jax.experimental import pallas as pl

_NUM_CLASSES = 80
_FPN_STRIDES = (8.0, 16.0, 32.0, 64.0, 128.0)
_IMG = 512.0
_K = 1000
_NCAND = 5 * _K
_NPAD = 5120
_POST = 50
_NOUT = 64
_PRE_NMS_THRESH = 0.3
_NMS_THRESH = 0.45


def _score_kernel(cls_ref, ctr_ref, out_ref):
    cls_p = jax.nn.sigmoid(cls_ref[...])
    ctr_p = jax.nn.sigmoid(ctr_ref[...])
    out_ref[...] = jnp.where(cls_p > _PRE_NMS_THRESH, cls_p * ctr_p, 0.0)


def _scores(cls_f, ctr_f):
    return pl.pallas_call(
        _score_kernel,
        out_shape=jax.ShapeDtypeStruct(cls_f.shape, jnp.float32),
    )(cls_f, ctr_f)


_ROWS = 8
_COLS = _NPAD // _ROWS


def _nms_kernel(r_ref, loc_ref, v_ref, c_ref, bo_ref, so_ref, co_ref):
    r = r_ref[0]
    loc = loc_ref[0]
    v = v_ref[0]
    cf = c_ref[0]

    sh = (_ROWS, _COLS)
    lane = (jax.lax.broadcasted_iota(jnp.int32, sh, 0) * _COLS
            + jax.lax.broadcasted_iota(jnp.int32, sh, 1))

    stride = jnp.where(
        lane < 1 * _K, _FPN_STRIDES[0],
        jnp.where(lane < 2 * _K, _FPN_STRIDES[1],
                  jnp.where(lane < 3 * _K, _FPN_STRIDES[2],
                            jnp.where(lane < 4 * _K, _FPN_STRIDES[3],
                                      _FPN_STRIDES[4]))))
    hsz = _IMG / stride

    xf = jnp.floor(loc / hsz)
    yf = loc - xf * hsz
    ysel = 0.5 * stride + yf * stride
    xsel = 0.5 * stride + xf * stride

    d = jnp.exp(r) * stride[None]
    hi = _IMG - 1.0
    x1 = jnp.clip(xsel - d[0], 0.0, hi)
    y1 = jnp.clip(ysel - d[1], 0.0, hi)
    x2 = jnp.clip(xsel + d[2], 0.0, hi)
    y2 = jnp.clip(ysel + d[3], 0.0, hi)

    valid = lane < _NCAND
    scores = jnp.where(valid, jnp.sqrt(jnp.maximum(v, 1e-12)), -1.0)

    off = cf * (2.0 * _IMG)
    x1o = x1 + off
    y1o = y1 + off
    x2o = x2 + off
    y2o = y2 + off
    area = (x2o - x1o) * (y2o - y1o)

    olane = jax.lax.broadcasted_iota(jnp.int32, (1, _NOUT), 1)
    zo = jnp.zeros((1, _NOUT), jnp.float32)

    def body(i, carry):
        s, ox1, oy1, ox2, oy2, osc, ocl = carry
        m = jnp.max(s)
        best = jnp.min(jnp.where(s == m, lane, _NPAD))
        msk = lane == best
        bcl = jnp.sum(jnp.where(msk, cf, 0.0))
        bsc = jnp.sum(jnp.where(msk, scores, 0.0))
        bx1 = jnp.sum(jnp.where(msk, x1o, 0.0))
        by1 = jnp.sum(jnp.where(msk, y1o, 0.0))
        bx2 = jnp.sum(jnp.where(msk, x2o, 0.0))
        by2 = jnp.sum(jnp.where(msk, y2o, 0.0))
        ix1 = jnp.maximum(bx1, x1o)
        iy1 = jnp.maximum(by1, y1o)
        ix2 = jnp.minimum(bx2, x2o)
        iy2 = jnp.minimum(by2, y2o)
        inter = jnp.maximum(ix2 - ix1, 0.0) * jnp.maximum(iy2 - iy1, 0.0)
        barea = (bx2 - bx1) * (by2 - by1)
        iou = inter / jnp.maximum(barea + area - inter, 1e-6)
        s = jnp.where(iou > _NMS_THRESH, -1.0, s)
        s = jnp.where(msk, -1.0, s)
        sel = olane == i
        boff = bcl * (2.0 * _IMG)
        ox1 = jnp.where(sel, bx1 - boff, ox1)
        oy1 = jnp.where(sel, by1 - boff, oy1)
        ox2 = jnp.where(sel, bx2 - boff, ox2)
        oy2 = jnp.where(sel, by2 - boff, oy2)
        osc = jnp.where(sel, bsc, osc)
        ocl = jnp.where(sel, bcl, ocl)
        return (s, ox1, oy1, ox2, oy2, osc, ocl)

    init = (scores, zo, zo, zo, zo, zo, zo)
    _, ox1, oy1, ox2, oy2, osc, ocl = jax.lax.fori_loop(0, _POST, body, init)

    bo_ref[0] = jnp.concatenate([ox1, oy1, ox2, oy2], axis=0)
    so_ref[0] = osc
    co_ref[0] = ocl


def kernel(images, cls0, cls1, cls2, cls3, cls4,
           reg0, reg1, reg2, reg3, reg4,
           ctr0, ctr1, ctr2, ctr3, ctr4):
    b = images.shape[0]
    cls_list = [cls0, cls1, cls2, cls3, cls4]
    reg_list = [reg0, reg1, reg2, reg3, reg4]
    ctr_list = [ctr0, ctr1, ctr2, ctr3, ctr4]

    rs, ls, vs, cs = [], [], [], []
    for lvl in range(5):
        cls = cls_list[lvl]
        _, C, h, w = cls.shape
        n = h * w
        cls_f = jnp.transpose(cls, (0, 3, 2, 1)).reshape(b, n, C)
        ctr_f = jnp.transpose(ctr_list[lvl], (0, 3, 2, 1)).reshape(b, n, 1)
        reg_f = jnp.transpose(reg_list[lvl], (0, 3, 2, 1)).reshape(b, n, 4)
        sc = _scores(cls_f, ctr_f).reshape(b, n * C)
        topv, topi = jax.lax.top_k(sc, _K)
        loc_idx = topi // C
        cls_idx = topi % C
        rsel = jnp.take_along_axis(reg_f, loc_idx[..., None], axis=1)
        rs.append(rsel)
        ls.append(loc_idx)
        vs.append(topv)
        cs.append(cls_idx)

    rsel = jnp.concatenate(rs, axis=1)
    loc = jnp.concatenate(ls, axis=1).astype(jnp.float32)
    topv = jnp.concatenate(vs, axis=1)
    clsf = jnp.concatenate(cs, axis=1).astype(jnp.float32)

    pad = _NPAD - _NCAND
    r4 = jnp.pad(jnp.transpose(rsel, (0, 2, 1)),
                 ((0, 0), (0, 0), (0, pad))).reshape(b, 4, _ROWS, _COLS)
    locp = jnp.pad(loc, ((0, 0), (0, pad))).reshape(b, _ROWS, _COLS)
    vp = jnp.pad(topv, ((0, 0), (0, pad))).reshape(b, _ROWS, _COLS)
    cp = jnp.pad(clsf, ((0, 0), (0, pad))).reshape(b, _ROWS, _COLS)

    bo, so, co = pl.pallas_call(
        _nms_kernel,
        grid=(b,),
        in_specs=[
            pl.BlockSpec((1, 4, _ROWS, _COLS), lambda i: (i, 0, 0, 0)),
            pl.BlockSpec((1, _ROWS, _COLS), lambda i: (i, 0, 0)),
            pl.BlockSpec((1, _ROWS, _COLS), lambda i: (i, 0, 0)),
            pl.BlockSpec((1, _ROWS, _COLS), lambda i: (i, 0, 0)),
        ],
        out_specs=[
            pl.BlockSpec((1, 4, _NOUT), lambda i: (i, 0, 0)),
            pl.BlockSpec((1, 1, _NOUT), lambda i: (i, 0, 0)),
            pl.BlockSpec((1, 1, _NOUT), lambda i: (i, 0, 0)),
        ],
        out_shape=[
            jax.ShapeDtypeStruct((b, 4, _NOUT), jnp.float32),
            jax.ShapeDtypeStruct((b, 1, _NOUT), jnp.float32),
            jax.ShapeDtypeStruct((b, 1, _NOUT), jnp.float32),
        ],
    )(r4, locp, vp, cp)

    pb = jnp.transpose(bo, (0, 2, 1))[:, :_POST, :]
    ps = so[:, 0, :_POST]
    pc = co[:, 0, :_POST].astype(jnp.int32)
    return pb, ps, pc
